# baseline (speedup 1.0000x reference)
import functools

import jax
import jax.numpy as jnp
from jax import lax
from jax.experimental import pallas as pl
from jax.experimental.pallas import tpu as pltpu
from jax.experimental.pallas import tpu_sc as plsc

L = 16


def _bcast_lane(v, jvec):
    return v.at[jvec].get(mode="promise_in_bounds")


@functools.lru_cache(maxsize=None)
def _build_sc_scatter(M, D, N):
    NW = 32
    assert N % L == 0
    SP = ((M + NW - 1) // NW + 7) // 8 * 8
    SL = M - (NW - 1) * SP
    assert 0 < SL <= SP and SP < (1 << 15)
    SCAN_STEPS = N // L
    MAP_WORDS = ((SP + L - 1) // L) * L
    CH = 488

    mesh = plsc.VectorSubcoreMesh(core_axis_name="c", subcore_axis_name="s")

    @functools.partial(
        pl.kernel,
        out_type=jax.ShapeDtypeStruct((M, D), jnp.float32),
        mesh=mesh,
        scratch_types=[
            pltpu.VMEM((N,), jnp.int32),
            pltpu.VMEM((N,), jnp.int32),
            pltpu.VMEM((MAP_WORDS,), jnp.int32),
            pltpu.VMEM((L, D), jnp.float32),
            pltpu.VMEM((CH, D), jnp.float32),
            pltpu.VMEM((CH, D), jnp.float32),
            pltpu.SemaphoreType.DMA,
            pltpu.SemaphoreType.DMA,
            pltpu.SemaphoreType.DMA,
            pltpu.SemaphoreType.DMA,
            pltpu.SemaphoreType.DMA,
            pltpu.SemaphoreType.DMA,
        ],
        compiler_params=pltpu.CompilerParams(use_tc_tiling_on_sc=False,
                                             needs_layout_passes=False),
    )
    def k(in_hbm, idx_hbm, val_hbm, out_hbm, idx_v, list_v, map_v, rows_v,
          bufa, bufb, idx_sem, gather_sem, ia_sem, ib_sem, oa_sem, ob_sem):
        wid = lax.axis_index("s") * 2 + lax.axis_index("c")
        lo = wid * SP
        last = wid == NW - 1
        S = jnp.where(last, SL, SP)

        idx_cp = pltpu.make_async_copy(idx_hbm, idx_v, idx_sem)
        idx_cp.start()

        bufs = (bufa, bufb)
        in_sems = (ia_sem, ib_sem)
        out_sems = (oa_sem, ob_sem)

        def sweep(nch, rem):
            chunks = [(c * CH, CH) for c in range(nch)]
            if rem:
                chunks.append((nch * CH, rem))
            n = len(chunks)
            for c, (off, rows) in enumerate(chunks):
                b = c % 2
                buf = bufs[b].at[pl.ds(0, rows)] if rows != CH else bufs[b]
                if c >= 2:
                    poff, prows = chunks[c - 2]
                    pbuf = (bufs[b].at[pl.ds(0, prows)] if prows != CH
                            else bufs[b])
                    pltpu.make_async_copy(
                        pbuf, out_hbm.at[pl.ds(lo + poff, prows)],
                        out_sems[b]).wait()
                cin = pltpu.make_async_copy(
                    in_hbm.at[pl.ds(lo + off, rows)], buf, in_sems[b])
                cin.start()
                cin.wait()
                pltpu.make_async_copy(
                    buf, out_hbm.at[pl.ds(lo + off, rows)],
                    out_sems[b]).start()
            for c in range(max(0, n - 2), n):
                b = c % 2
                off, rows = chunks[c]
                buf = bufs[b].at[pl.ds(0, rows)] if rows != CH else bufs[b]
                pltpu.make_async_copy(
                    buf, out_hbm.at[pl.ds(lo + off, rows)],
                    out_sems[b]).wait()

        @pl.when(jnp.logical_not(last))
        def _():
            sweep(SP // CH, SP % CH)

        @pl.when(last)
        def _():
            sweep(SL // CH, SL % CH)

        idx_cp.wait()

        zeros = jnp.zeros((L,), jnp.int32)

        def zbody(i, carry):
            map_v[pl.ds(i * L, L)] = zeros
            return carry

        lax.fori_loop(0, MAP_WORDS // L, zbody, 0, unroll=4)

        iota = lax.iota(jnp.int32, L)

        def scan_body(k_, count):
            v = idx_v[pl.ds(k_ * L, L)]
            rel = v - lo
            m = (rel >= 0) & (rel < S)
            pos = k_ * L + iota
            packed = (rel << 15) | pos
            cs = plsc.cumsum(jnp.where(m, 1, 0))
            dest = count + cs - 1
            plsc.store_scatter(list_v, [dest], packed, mask=m)
            return count + jnp.max(cs)

        count = lax.fori_loop(0, SCAN_STEPS, scan_body, jnp.int32(0),
                              unroll=4)

        ngroups = (count + L - 1) // L
        ones = jnp.ones((L,), jnp.int32)

        def group_body(i, carry):
            g = ngroups - 1 - i
            base = g * L
            packed = list_v[pl.ds(base, L)]
            lanepos = base + iota
            tail_ok = lanepos < count
            rel = packed >> 15
            pos = packed & 0x7FFF
            tv = jnp.where(tail_ok, 1, 0)

            wp = pos
            for j in range(L):
                jv = jnp.full((L,), j, jnp.int32)
                rj = _bcast_lane(rel, jv)
                pj = _bcast_lane(pos, jv)
                tj = _bcast_lane(tv, jv)
                same = (rel == rj) & (tj > 0) & tail_ok
                wp = jnp.where(same, jnp.maximum(wp, pj), wp)

            relc = jnp.clip(rel, 0, SP - 1)
            claimed = plsc.load_gather(map_v, [relc], mask=tail_ok)
            fresh = tail_ok & (claimed == 0)
            plsc.store_scatter(map_v, [relc], ones, mask=fresh)

            nfresh = jnp.max(plsc.all_reduce_population_count(fresh))

            @pl.when(nfresh > 0)
            def _():
                j0 = plsc.all_reduce_ffs(fresh)
                widx = jnp.where(fresh, relc + lo, _bcast_lane(relc, j0) + lo)
                wpos = jnp.where(fresh, wp, _bcast_lane(wp, j0))
                wpos = jnp.clip(wpos, 0, N - 1)
                gcp = pltpu.make_async_copy(val_hbm.at[wpos], rows_v,
                                            gather_sem)
                gcp.start()
                gcp.wait()
                pltpu.sync_copy(rows_v, out_hbm.at[widx])

            return carry

        lax.fori_loop(0, ngroups, group_body, 0)

    return k


def kernel(input, indices, values, accumulate):
    M, D = input.shape
    idx_flat = indices.reshape(-1)
    val_flat = values.reshape(-1, D)
    N = idx_flat.shape[0]
    k = _build_sc_scatter(M, D, N)
    return k(input, idx_flat, val_flat)


# speedup vs baseline: 1.9664x; 1.9664x over previous
import functools

import jax
import jax.numpy as jnp
from jax import lax
from jax.experimental import pallas as pl
from jax.experimental.pallas import tpu as pltpu
from jax.experimental.pallas import tpu_sc as plsc

L = 16


@functools.lru_cache(maxsize=None)
def _build_sc_scatter(M, D, N):
    NW = 32
    MB = (M // 128) * 128
    CH = 896
    SP = 31360
    SL = MB - (NW - 1) * SP
    assert SP % CH == 0 and SP % 128 == 0 and 0 < SL <= SP
    NCH_F, REM_F = divmod(SP, CH)
    NCH_L, REM_L = divmod(SL, CH)
    assert REM_F == 0 and REM_L == 0
    assert SP < (1 << 15) and N <= (1 << 15)
    SCAN_STEPS = N // L

    mesh = plsc.VectorSubcoreMesh(core_axis_name="c", subcore_axis_name="s")

    @functools.partial(
        pl.kernel,
        out_type=jax.ShapeDtypeStruct((D, M), jnp.float32),
        mesh=mesh,
        scratch_types=[
            pltpu.VMEM((N,), jnp.int32),
            pltpu.VMEM((N,), jnp.int32),
            pltpu.VMEM((L, 128), jnp.float32),
            pltpu.VMEM((D, CH), jnp.float32),
            pltpu.VMEM((D, CH), jnp.float32),
            pltpu.SemaphoreType.DMA,
            pltpu.SemaphoreType.DMA,
            pltpu.SemaphoreType.DMA,
            pltpu.SemaphoreType.DMA,
            pltpu.SemaphoreType.DMA,
        ],
        compiler_params=pltpu.CompilerParams(needs_layout_passes=False),
    )
    def k(in_hbm, idx_hbm, val_hbm, out_hbm, idx_v, list_v, rows_v,
          buf0, buf1, gsem, i0, i1, o0, o1):
        iota = lax.iota(jnp.int32, L)
        wid = lax.axis_index("s") * 2 + lax.axis_index("c")
        lo = wid * SP
        last = wid == NW - 1
        S = jnp.where(last, SL, SP)

        bufs = (buf0, buf1)
        in_sems = (i0, i1)
        out_sems = (o0, o1)

        pltpu.sync_copy(idx_hbm, idx_v)

        def scan_body(k_, count):
            v = idx_v[pl.ds(k_ * L, L)]
            rel = v - lo
            m = (rel >= 0) & (rel < S)
            pos = k_ * L + iota
            packed = (rel << 15) | pos
            cs = plsc.cumsum(jnp.where(m, 1, 0))
            dest = count + cs - 1
            plsc.store_scatter(list_v, [dest], packed, mask=m)
            return count + jnp.max(cs)

        count = lax.fori_loop(0, SCAN_STEPS, scan_body, jnp.int32(0),
                              unroll=4)
        ngroups = (count + L - 1) // L
        clist = idx_v

        def filter_chunk(c_rel_lo, span):
            def fbody(g, ccount):
                e = list_v[pl.ds(g * L, L)]
                tok = (g * L + iota) < count
                rel = e >> 15
                m = tok & (rel >= c_rel_lo) & (rel < c_rel_lo + span)
                cs = plsc.cumsum(jnp.where(m, 1, 0))
                dest = ccount + cs - 1
                plsc.store_scatter(clist, [dest], e, mask=m)
                return ccount + jnp.max(cs)

            return lax.fori_loop(0, ngroups, fbody, jnp.int32(0))

        def apply_chunk(buf, c_rel_lo, ccount):
            cgroups = (ccount + L - 1) // L

            def abody(h, carry):
                e = clist[pl.ds(h * L, L)]
                tok = (h * L + iota) < ccount
                relv = e >> 15
                posv = e & 0x7FFF
                posc = jnp.where(tok, posv, 0)
                gcp = pltpu.make_async_copy(
                    val_hbm.at[posc >> 2], rows_v, gsem)
                gcp.start()
                gcp.wait()
                colv = relv - c_rel_lo
                cnt_rem = ccount - h * L
                for j in range(L):
                    @pl.when(j < cnt_rem)
                    def _():
                        cvec = jnp.broadcast_to(colv[j], (L,))
                        q = posv[j] & 3
                        dlo = rows_v[j, pl.ds(q * 32, L)]
                        dhi = rows_v[j, pl.ds(q * 32 + L, L)]
                        plsc.store_scatter(buf, [iota, cvec], dlo)
                        plsc.store_scatter(buf, [iota + L, cvec], dhi)
                return carry

            lax.fori_loop(0, cgroups, abody, 0)

        def sweep(n):
            def in_cp(b, c):
                return pltpu.make_async_copy(
                    in_hbm.at[:, pl.ds(lo + c * CH, CH)], bufs[b],
                    in_sems[b])

            def out_cp(b, c):
                return pltpu.make_async_copy(
                    bufs[b], out_hbm.at[:, pl.ds(lo + c * CH, CH)],
                    out_sems[b])

            in_cp(0, 0).start()
            T = (n + 1) // 2

            def obody(t, carry):
                c0 = t * 2
                for b in range(2):
                    c = c0 + b

                    @pl.when(c < n)
                    def _():
                        in_cp(b, c).wait()

                        @pl.when(c + 1 < n)
                        def _():
                            @pl.when(c >= 1)
                            def _():
                                out_cp((b + 1) % 2, c - 1).wait()

                            in_cp((b + 1) % 2, c + 1).start()

                        ccount = filter_chunk(c * CH, CH)

                        @pl.when(ccount > 0)
                        def _():
                            apply_chunk(bufs[b], c * CH, ccount)

                        out_cp(b, c).start()

                return carry

            lax.fori_loop(0, T, obody, 0)
            for c_last in (n - 2, n - 1):
                if c_last >= 0:
                    out_cp(c_last % 2, c_last).wait()

        @pl.when(jnp.logical_not(last))
        def _():
            sweep(NCH_F)

        @pl.when(last)
        def _():
            sweep(NCH_L)

    return k


def kernel(input, indices, values, accumulate):
    M, D = input.shape
    idx_flat = indices.reshape(-1)
    val_flat = values.reshape(-1, D)
    N = idx_flat.shape[0]
    assert D == 32 and N % 4 == 0

    k = _build_sc_scatter(M, D, N)
    val_rs = val_flat.reshape(N // 4, 4 * D)
    out_t = k(input.T, idx_flat, val_rs)
    out = out_t.T

    MB = (M // 128) * 128
    TAIL = M - MB
    if TAIL:
        pos = jnp.arange(N, dtype=jnp.int32)[:, None]
        match = idx_flat[:, None] == (MB + jnp.arange(TAIL, dtype=jnp.int32))
        lastpos = jnp.max(jnp.where(match, pos, -1), axis=0)
        has = lastpos >= 0
        tail_rows = jnp.where(has[:, None],
                              val_flat[jnp.clip(lastpos, 0, N - 1)],
                              input[MB:])
        out = lax.dynamic_update_slice(out, tail_rows, (MB, 0))
    return out


# speedup vs baseline: 1.9794x; 1.0067x over previous
import functools

import jax
import jax.numpy as jnp
from jax import lax
from jax.experimental import pallas as pl
from jax.experimental.pallas import tpu as pltpu
from jax.experimental.pallas import tpu_sc as plsc

L = 16


@functools.lru_cache(maxsize=None)
def _build_sc_scatter(M, D, N):
    NW = 32
    MB = (M // 128) * 128
    CH = 896
    SP = 31360
    SL = MB - (NW - 1) * SP
    assert SP % CH == 0 and SP % 128 == 0 and 0 < SL <= SP
    NCH_F, REM_F = divmod(SP, CH)
    NCH_L, REM_L = divmod(SL, CH)
    assert REM_F == 0 and REM_L == 0
    assert SP < (1 << 15) and N <= (1 << 15)
    SCAN_STEPS = N // L

    mesh = plsc.VectorSubcoreMesh(core_axis_name="c", subcore_axis_name="s")

    @functools.partial(
        pl.kernel,
        out_type=jax.ShapeDtypeStruct((D, M), jnp.float32),
        mesh=mesh,
        scratch_types=[
            pltpu.VMEM((N,), jnp.int32),
            pltpu.VMEM((N,), jnp.int32),
            pltpu.VMEM((L, 128), jnp.float32),
            pltpu.VMEM((D, CH), jnp.float32),
            pltpu.VMEM((D, CH), jnp.float32),
            pltpu.SemaphoreType.DMA,
            pltpu.SemaphoreType.DMA,
            pltpu.SemaphoreType.DMA,
            pltpu.SemaphoreType.DMA,
            pltpu.SemaphoreType.DMA,
        ],
        compiler_params=pltpu.CompilerParams(needs_layout_passes=False),
    )
    def k(in_hbm, idx_hbm, val_hbm, out_hbm, idx_v, list_v, rows_v,
          buf0, buf1, gsem, i0, i1, o0, o1):
        iota = lax.iota(jnp.int32, L)
        wid = lax.axis_index("s") * 2 + lax.axis_index("c")
        lo = wid * SP
        last = wid == NW - 1
        S = jnp.where(last, SL, SP)

        bufs = (buf0, buf1)
        in_sems = (i0, i1)
        out_sems = (o0, o1)

        pltpu.sync_copy(idx_hbm, idx_v)

        def scan_body(k_, count):
            v = idx_v[pl.ds(k_ * L, L)]
            rel = v - lo
            m = (rel >= 0) & (rel < S)
            pos = k_ * L + iota
            packed = (rel << 15) | pos
            cs = plsc.cumsum(jnp.where(m, 1, 0))
            dest = count + cs - 1
            plsc.store_scatter(list_v, [dest], packed, mask=m)
            pop = plsc.all_reduce_population_count(m)
            return count + pop[0]

        count = lax.fori_loop(0, SCAN_STEPS, scan_body, jnp.int32(0),
                              unroll=8)
        ngroups = (count + L - 1) // L
        clist = idx_v

        def filter_chunk(c_rel_lo, span):
            def fbody(g, ccount):
                e = list_v[pl.ds(g * L, L)]
                tok = (g * L + iota) < count
                rel = e >> 15
                m = tok & (rel >= c_rel_lo) & (rel < c_rel_lo + span)
                cs = plsc.cumsum(jnp.where(m, 1, 0))
                dest = ccount + cs - 1
                plsc.store_scatter(clist, [dest], e, mask=m)
                pop = plsc.all_reduce_population_count(m)
                return ccount + pop[0]

            return lax.fori_loop(0, ngroups, fbody, jnp.int32(0))

        def apply_chunk(buf, c_rel_lo, ccount):
            cgroups = (ccount + L - 1) // L

            def abody(h, carry):
                e = clist[pl.ds(h * L, L)]
                tok = (h * L + iota) < ccount
                relv = e >> 15
                posv = e & 0x7FFF
                posc = jnp.where(tok, posv, 0)
                gcp = pltpu.make_async_copy(
                    val_hbm.at[posc >> 2], rows_v, gsem)
                gcp.start()
                gcp.wait()
                colv = relv - c_rel_lo
                cnt_rem = ccount - h * L
                for j in range(L):
                    @pl.when(j < cnt_rem)
                    def _():
                        cvec = jnp.broadcast_to(colv[j], (L,))
                        q = posv[j] & 3
                        dlo = rows_v[j, pl.ds(q * 32, L)]
                        dhi = rows_v[j, pl.ds(q * 32 + L, L)]
                        plsc.store_scatter(buf, [iota, cvec], dlo)
                        plsc.store_scatter(buf, [iota + L, cvec], dhi)
                return carry

            lax.fori_loop(0, cgroups, abody, 0)

        def sweep(n):
            def in_cp(b, c):
                return pltpu.make_async_copy(
                    in_hbm.at[:, pl.ds(lo + c * CH, CH)], bufs[b],
                    in_sems[b])

            def out_cp(b, c):
                return pltpu.make_async_copy(
                    bufs[b], out_hbm.at[:, pl.ds(lo + c * CH, CH)],
                    out_sems[b])

            in_cp(0, 0).start()
            T = (n + 1) // 2

            def obody(t, carry):
                c0 = t * 2
                for b in range(2):
                    c = c0 + b

                    @pl.when(c < n)
                    def _():
                        in_cp(b, c).wait()

                        @pl.when(c + 1 < n)
                        def _():
                            @pl.when(c >= 1)
                            def _():
                                out_cp((b + 1) % 2, c - 1).wait()

                            in_cp((b + 1) % 2, c + 1).start()

                        ccount = filter_chunk(c * CH, CH)

                        @pl.when(ccount > 0)
                        def _():
                            apply_chunk(bufs[b], c * CH, ccount)

                        out_cp(b, c).start()

                return carry

            lax.fori_loop(0, T, obody, 0)
            for c_last in (n - 2, n - 1):
                if c_last >= 0:
                    out_cp(c_last % 2, c_last).wait()

        @pl.when(jnp.logical_not(last))
        def _():
            sweep(NCH_F)

        @pl.when(last)
        def _():
            sweep(NCH_L)

    return k


def kernel(input, indices, values, accumulate):
    M, D = input.shape
    idx_flat = indices.reshape(-1)
    val_flat = values.reshape(-1, D)
    N = idx_flat.shape[0]
    assert D == 32 and N % 4 == 0

    k = _build_sc_scatter(M, D, N)
    val_rs = val_flat.reshape(N // 4, 4 * D)
    out_t = k(input.T, idx_flat, val_rs)
    out = out_t.T

    MB = (M // 128) * 128
    TAIL = M - MB
    if TAIL:
        pos = jnp.arange(N, dtype=jnp.int32)[:, None]
        match = idx_flat[:, None] == (MB + jnp.arange(TAIL, dtype=jnp.int32))
        lastpos = jnp.max(jnp.where(match, pos, -1), axis=0)
        has = lastpos >= 0
        tail_rows = jnp.where(has[:, None],
                              val_flat[jnp.clip(lastpos, 0, N - 1)],
                              input[MB:])
        out = lax.dynamic_update_slice(out, tail_rows, (MB, 0))
    return out
